# baseline (speedup 1.0000x reference)
import functools

import jax
import jax.numpy as jnp
from jax import lax
from jax.experimental import pallas as pl
from jax.experimental.pallas import tpu as pltpu
from jax.experimental.pallas import tpu_sc as plsc

_B = 4096
_H = 200
_D = 32
_N = _B * _H
_NC, _NS = 2, 16
_NW = _NC * _NS
_N_PER_W = _N // _NW
_CHUNK = 1280
_NCHUNK = _N_PER_W // _CHUNK


def _sc_gather(idx_flat, table):
    mesh = plsc.VectorSubcoreMesh(core_axis_name="c", subcore_axis_name="s")

    @functools.partial(
        pl.kernel,
        mesh=mesh,
        compiler_params=pltpu.CompilerParams(use_tc_tiling_on_sc=False),
        out_type=jax.ShapeDtypeStruct((_N, _D), jnp.float32),
        scratch_types=[
            pltpu.VMEM((_N_PER_W,), jnp.int32),
            pltpu.VMEM((2 * _CHUNK, _D), jnp.float32),
            pltpu.SemaphoreType.DMA,
            pltpu.SemaphoreType.DMA,
            pltpu.SemaphoreType.DMA,
            pltpu.SemaphoreType.DMA,
        ],
    )
    def k(idx_hbm, table_hbm, out_hbm, idx_v, rows_v,
          sem_g0, sem_g1, sem_s0, sem_s1):
        wid = lax.axis_index("s") * _NC + lax.axis_index("c")
        base = wid * _N_PER_W

        pltpu.sync_copy(idx_hbm.at[pl.ds(base, _N_PER_W)], idx_v)

        def start_gather(c, off, sem):
            half = _CHUNK // 2
            for j in range(2):
                pltpu.async_copy(
                    table_hbm.at[
                        idx_v.at[pl.ds(c * _CHUNK + j * half, half)]],
                    rows_v.at[pl.ds(off + j * half, half)],
                    sem,
                )

        def wait_store(c, off, sem):
            pltpu.make_async_copy(
                rows_v.at[pl.ds(off, _CHUNK)],
                out_hbm.at[pl.ds(base + c * _CHUNK, _CHUNK)],
                sem,
            ).wait()

        start_gather(0, 0, sem_g0)

        def body(g, carry):
            nxt = g + 1
            even_n = lax.rem(nxt, 2) == 0
            even_g = lax.rem(g, 2) == 0

            @pl.when(jnp.logical_and(nxt < _NCHUNK, even_n))
            def _():
                wait_store(g - 1, 0, sem_s0)
                start_gather(nxt, 0, sem_g0)

            @pl.when(jnp.logical_and(nxt < _NCHUNK, jnp.logical_not(even_n)))
            def _():
                @pl.when(g >= 1)
                def _():
                    wait_store(g - 1, _CHUNK, sem_s1)

                start_gather(nxt, _CHUNK, sem_g1)

            @pl.when(even_g)
            def _():
                pltpu.make_async_copy(
                    out_hbm.at[pl.ds(0, _CHUNK)],
                    rows_v.at[pl.ds(0, _CHUNK)],
                    sem_g0,
                ).wait()
                pltpu.async_copy(
                    rows_v.at[pl.ds(0, _CHUNK)],
                    out_hbm.at[pl.ds(base + g * _CHUNK, _CHUNK)],
                    sem_s0,
                )

            @pl.when(jnp.logical_not(even_g))
            def _():
                pltpu.make_async_copy(
                    out_hbm.at[pl.ds(0, _CHUNK)],
                    rows_v.at[pl.ds(_CHUNK, _CHUNK)],
                    sem_g1,
                ).wait()
                pltpu.async_copy(
                    rows_v.at[pl.ds(_CHUNK, _CHUNK)],
                    out_hbm.at[pl.ds(base + g * _CHUNK, _CHUNK)],
                    sem_s1,
                )

            return carry

        lax.fori_loop(0, _NCHUNK, body, 0)

        pltpu.make_async_copy(
            rows_v.at[pl.ds(0, _CHUNK)],
            out_hbm.at[pl.ds(base + (_NCHUNK - 2) * _CHUNK, _CHUNK)],
            sem_s0,
        ).wait()
        pltpu.make_async_copy(
            rows_v.at[pl.ds(_CHUNK, _CHUNK)],
            out_hbm.at[pl.ds(base + (_NCHUNK - 1) * _CHUNK, _CHUNK)],
            sem_s1,
        ).wait()

    return k(idx_flat, table)


def kernel(input_seqs, table):
    idx_flat = input_seqs.reshape(_N).astype(jnp.int32)
    out = _sc_gather(idx_flat, table)
    return out.reshape(_B, _H, _D)


# speedup vs baseline: 1.0001x; 1.0001x over previous
import functools

import jax
import jax.numpy as jnp
from jax import lax
from jax.experimental import pallas as pl
from jax.experimental.pallas import tpu as pltpu
from jax.experimental.pallas import tpu_sc as plsc

_B = 4096
_H = 200
_D = 32
_N = _B * _H
_NC, _NS = 2, 16
_NW = _NC * _NS
_N_PER_W = _N // _NW
_CHUNK = 1280
_NCHUNK = _N_PER_W // _CHUNK


def _sc_gather(idx_flat, table):
    mesh = plsc.VectorSubcoreMesh(core_axis_name="c", subcore_axis_name="s")

    @functools.partial(
        pl.kernel,
        mesh=mesh,
        compiler_params=pltpu.CompilerParams(use_tc_tiling_on_sc=False),
        out_type=jax.ShapeDtypeStruct((_N, _D), jnp.float32),
        scratch_types=[
            pltpu.VMEM((_N_PER_W,), jnp.int32),
            pltpu.VMEM((2 * _CHUNK, _D), jnp.float32),
            pltpu.SemaphoreType.DMA,
            pltpu.SemaphoreType.DMA,
            pltpu.SemaphoreType.DMA,
            pltpu.SemaphoreType.DMA,
        ],
    )
    def k(idx_hbm, table_hbm, out_hbm, idx_v, rows_v,
          sem_g0, sem_g1, sem_s0, sem_s1):
        wid = lax.axis_index("s") * _NC + lax.axis_index("c")
        base = wid * _N_PER_W

        pltpu.sync_copy(idx_hbm.at[pl.ds(base, _N_PER_W)], idx_v)

        def start_gather(c, off, sem):
            pltpu.async_copy(
                table_hbm.at[idx_v.at[pl.ds(c * _CHUNK, _CHUNK)]],
                rows_v.at[pl.ds(off, _CHUNK)],
                sem,
            )

        def wait_store(c, off, sem):
            pltpu.make_async_copy(
                rows_v.at[pl.ds(off, _CHUNK)],
                out_hbm.at[pl.ds(base + c * _CHUNK, _CHUNK)],
                sem,
            ).wait()

        start_gather(0, 0, sem_g0)

        def body(g, carry):
            nxt = g + 1
            even_n = lax.rem(nxt, 2) == 0
            even_g = lax.rem(g, 2) == 0

            @pl.when(jnp.logical_and(nxt < _NCHUNK, even_n))
            def _():
                wait_store(g - 1, 0, sem_s0)
                start_gather(nxt, 0, sem_g0)

            @pl.when(jnp.logical_and(nxt < _NCHUNK, jnp.logical_not(even_n)))
            def _():
                @pl.when(g >= 1)
                def _():
                    wait_store(g - 1, _CHUNK, sem_s1)

                start_gather(nxt, _CHUNK, sem_g1)

            @pl.when(even_g)
            def _():
                pltpu.make_async_copy(
                    out_hbm.at[pl.ds(0, _CHUNK)],
                    rows_v.at[pl.ds(0, _CHUNK)],
                    sem_g0,
                ).wait()
                pltpu.async_copy(
                    rows_v.at[pl.ds(0, _CHUNK)],
                    out_hbm.at[pl.ds(base + g * _CHUNK, _CHUNK)],
                    sem_s0,
                )

            @pl.when(jnp.logical_not(even_g))
            def _():
                pltpu.make_async_copy(
                    out_hbm.at[pl.ds(0, _CHUNK)],
                    rows_v.at[pl.ds(_CHUNK, _CHUNK)],
                    sem_g1,
                ).wait()
                pltpu.async_copy(
                    rows_v.at[pl.ds(_CHUNK, _CHUNK)],
                    out_hbm.at[pl.ds(base + g * _CHUNK, _CHUNK)],
                    sem_s1,
                )

            return carry

        lax.fori_loop(0, _NCHUNK, body, 0)

        pltpu.make_async_copy(
            rows_v.at[pl.ds(0, _CHUNK)],
            out_hbm.at[pl.ds(base + (_NCHUNK - 2) * _CHUNK, _CHUNK)],
            sem_s0,
        ).wait()
        pltpu.make_async_copy(
            rows_v.at[pl.ds(_CHUNK, _CHUNK)],
            out_hbm.at[pl.ds(base + (_NCHUNK - 1) * _CHUNK, _CHUNK)],
            sem_s1,
        ).wait()

    return k(idx_flat, table)


def kernel(input_seqs, table):
    idx_flat = input_seqs.reshape(_N).astype(jnp.int32)
    out = _sc_gather(idx_flat, table)
    return out.reshape(_B, _H, _D)
